# Initial kernel scaffold; baseline (speedup 1.0000x reference)
import jax
import jax.numpy as jnp
from jax.experimental import pallas as pl


def kernel(features, superpixel_indices):
    raise NotImplementedError("write your pallas kernel here")


# speedup vs baseline: 2.6363x; 2.6363x over previous
import functools

import jax
import jax.numpy as jnp
from jax import lax
from jax.experimental import pallas as pl
from jax.experimental.pallas import tpu as pltpu
from jax.experimental.pallas import tpu_sc as plsc

_B, _C, _H, _W = 4, 96, 384, 384
_HW = _H * _W
_N = 32
_NC, _NS, _L = 2, 16, 16
_NW = _NC * _NS
_SH = _NW // _B
_PW = _HW // _SH
_GRP = _PW // _L
_UNROLL = 8


def _sc_segment_sums(feat, idx):
    mesh = plsc.VectorSubcoreMesh(core_axis_name="c", subcore_axis_name="s")

    @functools.partial(
        pl.kernel,
        out_type=(
            jax.ShapeDtypeStruct((_NW, _C * _N * _L), jnp.float32),
            jax.ShapeDtypeStruct((_NW, _N * _L), jnp.float32),
        ),
        mesh=mesh,
        scratch_types=[
            pltpu.VMEM((_PW,), jnp.int32),
            pltpu.VMEM((2, _PW), jnp.float32),
            pltpu.VMEM((_C * _N * _L,), jnp.float32),
            pltpu.VMEM((_N * _L,), jnp.float32),
            pltpu.SemaphoreType.DMA((2,)),
            pltpu.SemaphoreType.DMA,
        ],
        compiler_params=pltpu.CompilerParams(needs_layout_passes=False),
    )
    def seg_kernel(feat_hbm, idx_hbm, psum_hbm, pcnt_hbm,
                   idx_v, fbuf, acc, cacc, fsem, isem):
        wid = lax.axis_index("s") * _NC + lax.axis_index("c")
        b = wid // _SH
        sh = wid % _SH
        p0 = sh * _PW
        row0 = b * _C

        pltpu.async_copy(idx_hbm.at[b, pl.ds(p0, _PW)], idx_v, isem).wait()
        pltpu.async_copy(feat_hbm.at[row0, pl.ds(p0, _PW)],
                         fbuf.at[0], fsem.at[0])

        lanes = lax.iota(jnp.int32, _L)
        zeros = jnp.zeros((_L,), jnp.float32)
        ones = jnp.ones((_L,), jnp.float32)

        def zero_acc(r, carry):
            acc[pl.ds(r * _L, _L)] = zeros
            return carry
        lax.fori_loop(0, _C * _N, zero_acc, 0)

        def zero_cacc(r, carry):
            cacc[pl.ds(r * _L, _L)] = zeros
            return carry
        lax.fori_loop(0, _N, zero_cacc, 0)

        def count_grp(g, carry):
            iv = idx_v[pl.ds(g * _L, _L)]
            plsc.addupdate_scatter(cacc, [iv * _L + lanes], ones)
            return carry
        lax.fori_loop(0, _GRP, count_grp, 0)

        def chan_body(c, carry):
            slot = lax.rem(c, 2)
            nslot = 1 - slot

            @pl.when(c + 1 < _C)
            def _prefetch():
                pltpu.async_copy(feat_hbm.at[row0 + c + 1, pl.ds(p0, _PW)],
                                 fbuf.at[nslot], fsem.at[nslot])

            pltpu.make_async_copy(feat_hbm.at[row0, pl.ds(p0, _PW)],
                                  fbuf.at[slot], fsem.at[slot]).wait()
            lanesb = lanes + c * (_N * _L)

            def grp(g, inner):
                g0 = g * (_L * _UNROLL)
                for u in range(_UNROLL):
                    off = g0 + u * _L
                    iv = idx_v[pl.ds(off, _L)]
                    v = fbuf[slot, pl.ds(off, _L)]
                    plsc.addupdate_scatter(acc, [iv * _L + lanesb], v)
                return inner
            lax.fori_loop(0, _GRP // _UNROLL, grp, 0)
            return carry
        lax.fori_loop(0, _C, chan_body, 0)

        pltpu.sync_copy(acc, psum_hbm.at[wid])
        pltpu.sync_copy(cacc, pcnt_hbm.at[wid])

    return seg_kernel(feat, idx)


def _tc_finalize(psum, pcnt):
    def body(ps_ref, pc_ref, spt_ref, sim_ref):
        x = jnp.sum(ps_ref[0], axis=0)
        cn = jnp.sum(pc_ref[0], axis=0)
        col = lax.broadcasted_iota(jnp.int32, (_N * _L, _N), 0) // _L
        seg = lax.broadcasted_iota(jnp.int32, (_N * _L, _N), 1)
        onehot = (col == seg).astype(jnp.float32)
        dims = (((1,), (0,)), ((), ()))
        sums = lax.dot_general(x, onehot, dims,
                               precision=lax.Precision.HIGHEST)
        counts = lax.dot_general(cn[None, :], onehot, dims,
                                 precision=lax.Precision.HIGHEST)
        m = sums / jnp.maximum(counts, 1.0)
        spt_ref[0] = m
        gram = lax.dot_general(m, m, (((0,), (0,)), ((), ())),
                               precision=lax.Precision.HIGHEST)
        r = jnp.sum(m * m, axis=0)
        sim_ref[0] = 1.0 - 0.5 * (r[:, None] + r[None, :]) + gram

    return pl.pallas_call(
        body,
        grid=(_B,),
        in_specs=[
            pl.BlockSpec((1, _SH, _C, _N * _L), lambda b: (b, 0, 0, 0)),
            pl.BlockSpec((1, _SH, _N * _L), lambda b: (b, 0, 0)),
        ],
        out_specs=[
            pl.BlockSpec((1, _C, _N), lambda b: (b, 0, 0)),
            pl.BlockSpec((1, _N, _N), lambda b: (b, 0, 0)),
        ],
        out_shape=[
            jax.ShapeDtypeStruct((_B, _C, _N), jnp.float32),
            jax.ShapeDtypeStruct((_B, _N, _N), jnp.float32),
        ],
    )(psum, pcnt)


def kernel(features, superpixel_indices):
    feat = features.reshape(_B * _C, _HW)
    idx = superpixel_indices.reshape(_B, _HW)
    psum, pcnt = _sc_segment_sums(feat, idx)
    ps = psum.reshape(_B, _SH, _C, _N * _L)
    pc = pcnt.reshape(_B, _SH, _N * _L)
    spt, sim = _tc_finalize(ps, pc)
    sp = jnp.transpose(spt, (0, 2, 1))
    return (sp, sim)


# speedup vs baseline: 10.1770x; 3.8604x over previous
import functools

import jax
import jax.numpy as jnp
from jax import lax
from jax.experimental import pallas as pl
from jax.experimental.pallas import tpu as pltpu
from jax.experimental.pallas import tpu_sc as plsc

_B, _C, _H, _W = 4, 96, 384, 384
_HW = _H * _W
_N = 32
_NL = _N * 16
_NC, _NS, _L = 2, 16, 16
_NW = _NC * _NS
_SH = _NW // _B
_ROWS = _H // _SH
_GPR = _W // _L


def _sc_segment_sums(feat, idx):
    mesh = plsc.VectorSubcoreMesh(core_axis_name="c", subcore_axis_name="s")

    @functools.partial(
        pl.kernel,
        out_type=(
            jax.ShapeDtypeStruct((_NW, _C * _NL), jnp.float32),
            jax.ShapeDtypeStruct((_NW, _NL), jnp.float32),
        ),
        mesh=mesh,
        scratch_types=[
            pltpu.VMEM((_ROWS, _W), jnp.int32),
            pltpu.VMEM((_ROWS, _W), jnp.float32),
            pltpu.VMEM((_ROWS, _W), jnp.float32),
            pltpu.VMEM((_C * _NL,), jnp.float32),
            pltpu.VMEM((_NL,), jnp.float32),
            pltpu.SemaphoreType.DMA,
            pltpu.SemaphoreType.DMA,
            pltpu.SemaphoreType.DMA,
        ],
        compiler_params=pltpu.CompilerParams(needs_layout_passes=False),
    )
    def seg_kernel(feat_hbm, idx_hbm, psum_hbm, pcnt_hbm,
                   ivf, fba, fbb, acc, cacc, sema, semb, isem):
        wid = lax.axis_index("s") * _NC + lax.axis_index("c")
        b = wid // _SH
        r0 = (wid % _SH) * _ROWS

        pltpu.async_copy(idx_hbm.at[b, pl.ds(r0, _ROWS), :], ivf, isem).wait()
        pltpu.async_copy(feat_hbm.at[b, 0, pl.ds(r0, _ROWS), :], fba, sema)
        pltpu.async_copy(feat_hbm.at[b, 1, pl.ds(r0, _ROWS), :], fbb, semb)

        lanes = lax.iota(jnp.int32, _L)
        zeros = jnp.zeros((_L,), jnp.float32)
        ones = jnp.ones((_L,), jnp.float32)

        @plsc.parallel_loop(0, _C * _N, unroll=8)
        def _zero_acc(r):
            acc[pl.ds(r * _L, _L)] = zeros

        @plsc.parallel_loop(0, _N, unroll=8)
        def _zero_cacc(r):
            cacc[pl.ds(r * _L, _L)] = zeros

        def _prep_row(r, carry):
            @plsc.parallel_loop(0, _GPR, unroll=8)
            def _grp(u):
                iv = ivf[r, pl.ds(u * _L, _L)]
                x = iv * _L + lanes
                ivf[r, pl.ds(u * _L, _L)] = x
                plsc.addupdate_scatter(cacc, [x], ones)
            return carry
        lax.fori_loop(0, _ROWS, _prep_row, 0)

        def _pair(kk, carry):
            ch0 = 2 * kk
            for choff, fbuf, sem in ((0, fba, sema), (1, fbb, semb)):
                ch = ch0 + choff
                pltpu.make_async_copy(
                    feat_hbm.at[b, ch, pl.ds(r0, _ROWS), :], fbuf, sem
                ).wait()
                accc = acc.at[pl.ds(ch * _NL, _NL)]

                def _row(r, inner, fbuf=fbuf, accc=accc):
                    @plsc.parallel_loop(0, _GPR, unroll=8)
                    def _grp(u):
                        x = ivf[r, pl.ds(u * _L, _L)]
                        v = fbuf[r, pl.ds(u * _L, _L)]
                        plsc.addupdate_scatter(accc, [x], v)
                    return inner
                lax.fori_loop(0, _ROWS, _row, 0)

                @pl.when(ch + 2 < _C)
                def _prefetch(ch=ch, fbuf=fbuf, sem=sem):
                    pltpu.async_copy(
                        feat_hbm.at[b, ch + 2, pl.ds(r0, _ROWS), :], fbuf, sem
                    )
            return carry
        lax.fori_loop(0, _C // 2, _pair, 0)

        pltpu.sync_copy(acc, psum_hbm.at[wid])
        pltpu.sync_copy(cacc, pcnt_hbm.at[wid])

    return seg_kernel(feat, idx)


def _tc_finalize(psum, pcnt):
    def body(ps_ref, pc_ref, spt_ref, sim_ref):
        x = jnp.sum(ps_ref[0], axis=0)
        cn = jnp.sum(pc_ref[0], axis=0)
        col = lax.broadcasted_iota(jnp.int32, (_NL, _N), 0) // _L
        seg = lax.broadcasted_iota(jnp.int32, (_NL, _N), 1)
        onehot = (col == seg).astype(jnp.float32)
        dims = (((1,), (0,)), ((), ()))
        sums = lax.dot_general(x, onehot, dims,
                               precision=lax.Precision.HIGHEST)
        counts = lax.dot_general(cn[None, :], onehot, dims,
                                 precision=lax.Precision.HIGHEST)
        m = sums / jnp.maximum(counts, 1.0)
        spt_ref[0] = m
        gram = lax.dot_general(m, m, (((0,), (0,)), ((), ())),
                               precision=lax.Precision.HIGHEST)
        r = jnp.sum(m * m, axis=0)
        sim_ref[0] = 1.0 - 0.5 * (r[:, None] + r[None, :]) + gram

    return pl.pallas_call(
        body,
        grid=(_B,),
        in_specs=[
            pl.BlockSpec((1, _SH, _C, _NL), lambda b: (b, 0, 0, 0)),
            pl.BlockSpec((1, _SH, _NL), lambda b: (b, 0, 0)),
        ],
        out_specs=[
            pl.BlockSpec((1, _C, _N), lambda b: (b, 0, 0)),
            pl.BlockSpec((1, _N, _N), lambda b: (b, 0, 0)),
        ],
        out_shape=[
            jax.ShapeDtypeStruct((_B, _C, _N), jnp.float32),
            jax.ShapeDtypeStruct((_B, _N, _N), jnp.float32),
        ],
    )(psum, pcnt)


def kernel(features, superpixel_indices):
    psum, pcnt = _sc_segment_sums(features, superpixel_indices)
    ps = psum.reshape(_B, _SH, _C, _NL)
    pc = pcnt.reshape(_B, _SH, _NL)
    spt, sim = _tc_finalize(ps, pc)
    sp = jnp.transpose(spt, (0, 2, 1))
    return (sp, sim)


# speedup vs baseline: 12.6579x; 1.2438x over previous
import functools

import jax
import jax.numpy as jnp
from jax import lax
from jax.experimental import pallas as pl
from jax.experimental.pallas import tpu as pltpu
from jax.experimental.pallas import tpu_sc as plsc

_B, _C, _H, _W = 4, 96, 384, 384
_N = 32
_L = 16
_NL = _N * _L
_NC, _NS = 2, 16
_NW = _NC * _NS
_SH = _NW // _B
_ROWS = _H // _SH
_HR = _ROWS // 2
_GPR = _W // _L


def _sc_segment_sums(feat, idx):
    mesh = plsc.VectorSubcoreMesh(core_axis_name="c", subcore_axis_name="s")

    @functools.partial(
        pl.kernel,
        out_type=(
            jax.ShapeDtypeStruct((_NW, _C * _NL), jnp.float32),
            jax.ShapeDtypeStruct((_NW, _NL), jnp.float32),
        ),
        mesh=mesh,
        scratch_types=[
            pltpu.VMEM((_ROWS, _W), jnp.int32),
            pltpu.VMEM((_HR, _W), jnp.float32),
            pltpu.VMEM((_HR, _W), jnp.float32),
            pltpu.VMEM((_HR, _W), jnp.float32),
            pltpu.VMEM((_HR, _W), jnp.float32),
            pltpu.VMEM((_C * _NL,), jnp.float32),
            pltpu.VMEM((_NL,), jnp.float32),
            pltpu.SemaphoreType.DMA,
            pltpu.SemaphoreType.DMA,
            pltpu.SemaphoreType.DMA,
            pltpu.SemaphoreType.DMA,
            pltpu.SemaphoreType.DMA,
        ],
        compiler_params=pltpu.CompilerParams(needs_layout_passes=False),
    )
    def seg_kernel(feat_hbm, idx_hbm, psum_hbm, pcnt_hbm,
                   ivf, fa0, fb0, fa1, fb1, acc, cacc,
                   sa0, sb0, sa1, sb1, isem):
        wid = lax.axis_index("s") * _NC + lax.axis_index("c")
        b = wid // _SH
        sh = wid % _SH
        r0 = sh * _ROWS

        pltpu.async_copy(idx_hbm.at[b, pl.ds(r0, _ROWS), :], ivf, isem).wait()
        pltpu.async_copy(feat_hbm.at[b, 0, pl.ds(r0, _HR), :], fa0, sa0)
        pltpu.async_copy(feat_hbm.at[b, 1, pl.ds(r0, _HR), :], fb0, sb0)
        pltpu.async_copy(feat_hbm.at[b, 0, pl.ds(r0 + _HR, _HR), :], fa1, sa1)
        pltpu.async_copy(feat_hbm.at[b, 1, pl.ds(r0 + _HR, _HR), :], fb1, sb1)

        lanes = lax.iota(jnp.int32, _L)
        zeros = jnp.zeros((_L,), jnp.float32)
        ones = jnp.ones((_L,), jnp.float32)

        def _zero_row(r, carry):
            @plsc.parallel_loop(0, _NL // _L, unroll=8)
            def _z(u):
                acc[pl.ds(r * _NL + u * _L, _L)] = zeros
            return carry
        lax.fori_loop(0, _C, _zero_row, 0)

        @plsc.parallel_loop(0, _NL // _L, unroll=8)
        def _zero_cacc(u):
            cacc[pl.ds(u * _L, _L)] = zeros

        def _prep_row(r, carry):
            @plsc.parallel_loop(0, _GPR, unroll=8)
            def _grp(u):
                iv = ivf[r, pl.ds(u * _L, _L)]
                x = iv * _L + lanes
                ivf[r, pl.ds(u * _L, _L)] = x
                plsc.addupdate_scatter(cacc, [x], ones)
            return carry
        lax.fori_loop(0, _ROWS, _prep_row, 0)

        def _pair(kk, carry):
            ch0 = 2 * kk
            for half, fa, fb, sema, semb in (
                (0, fa0, fb0, sa0, sb0),
                (1, fa1, fb1, sa1, sb1),
            ):
                rbase = half * _HR
                pltpu.make_async_copy(
                    feat_hbm.at[b, ch0, pl.ds(r0 + rbase, _HR), :], fa, sema
                ).wait()
                pltpu.make_async_copy(
                    feat_hbm.at[b, ch0 + 1, pl.ds(r0 + rbase, _HR), :], fb,
                    semb,
                ).wait()
                acc0 = acc.at[pl.ds(ch0 * _NL, _NL)]
                acc1 = acc.at[pl.ds((ch0 + 1) * _NL, _NL)]

                def _row(r, inner, fa=fa, fb=fb, rbase=rbase,
                         acc0=acc0, acc1=acc1):
                    @plsc.parallel_loop(0, _GPR, unroll=8)
                    def _grp(u):
                        x = ivf[r + rbase, pl.ds(u * _L, _L)]
                        va = fa[r, pl.ds(u * _L, _L)]
                        vb = fb[r, pl.ds(u * _L, _L)]
                        plsc.addupdate_scatter(acc0, [x], va)
                        plsc.addupdate_scatter(acc1, [x], vb)
                    return inner
                lax.fori_loop(0, _HR, _row, 0)

                @pl.when(ch0 + 2 < _C)
                def _prefetch(ch0=ch0, rbase=rbase, fa=fa, fb=fb,
                              sema=sema, semb=semb):
                    pltpu.async_copy(
                        feat_hbm.at[b, ch0 + 2, pl.ds(r0 + rbase, _HR), :],
                        fa, sema,
                    )
                    pltpu.async_copy(
                        feat_hbm.at[b, ch0 + 3, pl.ds(r0 + rbase, _HR), :],
                        fb, semb,
                    )
            return carry
        lax.fori_loop(0, _C // 2, _pair, 0)

        pltpu.sync_copy(acc, psum_hbm.at[wid])
        pltpu.sync_copy(cacc, pcnt_hbm.at[wid])

    return seg_kernel(feat, idx)


def _tc_finalize(psum, pcnt):
    def body(ps_ref, pc_ref, sp_ref, sim_ref):
        x = jnp.sum(ps_ref[0], axis=0)
        cn = jnp.sum(pc_ref[0], axis=0)
        col = lax.broadcasted_iota(jnp.int32, (_NL, _N), 0) // _L
        seg = lax.broadcasted_iota(jnp.int32, (_NL, _N), 1)
        onehot = (col == seg).astype(jnp.float32)
        dims = (((0,), (1,)), ((), ()))
        sums_t = lax.dot_general(onehot, x, dims,
                                 precision=lax.Precision.HIGHEST)
        cnt_t = lax.dot_general(onehot, cn[None, :], dims,
                                precision=lax.Precision.HIGHEST)
        m = sums_t / jnp.maximum(cnt_t, 1.0)
        sp_ref[0] = m
        gram = lax.dot_general(m, m, (((1,), (1,)), ((), ())),
                               precision=lax.Precision.HIGHEST)
        r = jnp.sum(m * m, axis=1)
        sim_ref[0] = 1.0 - 0.5 * (r[:, None] + r[None, :]) + gram

    return pl.pallas_call(
        body,
        grid=(_B,),
        in_specs=[
            pl.BlockSpec((1, _SH, _C, _NL), lambda b: (b, 0, 0, 0)),
            pl.BlockSpec((1, _SH, _NL), lambda b: (b, 0, 0)),
        ],
        out_specs=[
            pl.BlockSpec((1, _N, _C), lambda b: (b, 0, 0)),
            pl.BlockSpec((1, _N, _N), lambda b: (b, 0, 0)),
        ],
        out_shape=[
            jax.ShapeDtypeStruct((_B, _N, _C), jnp.float32),
            jax.ShapeDtypeStruct((_B, _N, _N), jnp.float32),
        ],
    )(psum, pcnt)


def kernel(features, superpixel_indices):
    psum, pcnt = _sc_segment_sums(features, superpixel_indices)
    ps = psum.reshape(_B, _SH, _C, _NL)
    pc = pcnt.reshape(_B, _SH, _NL)
    sp, sim = _tc_finalize(ps, pc)
    return (sp, sim)


# speedup vs baseline: 12.7379x; 1.0063x over previous
import functools

import jax
import jax.numpy as jnp
from jax import lax
from jax.experimental import pallas as pl
from jax.experimental.pallas import tpu as pltpu
from jax.experimental.pallas import tpu_sc as plsc

_B, _C, _H, _W = 4, 96, 384, 384
_N = 32
_L = 16
_NL = _N * _L
_NC, _NS = 2, 16
_NW = _NC * _NS
_SH = _NW // _B
_ROWS = _H // _SH
_HR = _ROWS // 2
_GPR = _W // _L
_CS = 48
_CT = _C - _CS
_HB = 8


def _sc_segment_sums(feat, idx):
    mesh = plsc.VectorSubcoreMesh(core_axis_name="c", subcore_axis_name="s")

    @functools.partial(
        pl.kernel,
        out_type=(
            jax.ShapeDtypeStruct((_NW, _CS * _NL), jnp.float32),
            jax.ShapeDtypeStruct((_NW, _NL), jnp.float32),
        ),
        mesh=mesh,
        scratch_types=[
            pltpu.VMEM((_ROWS, _W), jnp.int32),
            pltpu.VMEM((_HR, _W), jnp.float32),
            pltpu.VMEM((_HR, _W), jnp.float32),
            pltpu.VMEM((_HR, _W), jnp.float32),
            pltpu.VMEM((_HR, _W), jnp.float32),
            pltpu.VMEM((_CS * _NL,), jnp.float32),
            pltpu.VMEM((_NL,), jnp.float32),
            pltpu.SemaphoreType.DMA,
            pltpu.SemaphoreType.DMA,
            pltpu.SemaphoreType.DMA,
            pltpu.SemaphoreType.DMA,
            pltpu.SemaphoreType.DMA,
        ],
        compiler_params=pltpu.CompilerParams(needs_layout_passes=False),
    )
    def seg_kernel(feat_hbm, idx_hbm, psum_hbm, pcnt_hbm,
                   ivf, fa0, fb0, fa1, fb1, acc, cacc,
                   sa0, sb0, sa1, sb1, isem):
        wid = lax.axis_index("s") * _NC + lax.axis_index("c")
        b = wid // _SH
        sh = wid % _SH
        r0 = sh * _ROWS

        pltpu.async_copy(idx_hbm.at[b, pl.ds(r0, _ROWS), :], ivf, isem).wait()
        pltpu.async_copy(feat_hbm.at[b, 0, pl.ds(r0, _HR), :], fa0, sa0)
        pltpu.async_copy(feat_hbm.at[b, 1, pl.ds(r0, _HR), :], fb0, sb0)
        pltpu.async_copy(feat_hbm.at[b, 0, pl.ds(r0 + _HR, _HR), :], fa1, sa1)
        pltpu.async_copy(feat_hbm.at[b, 1, pl.ds(r0 + _HR, _HR), :], fb1, sb1)

        lanes = lax.iota(jnp.int32, _L)
        zeros = jnp.zeros((_L,), jnp.float32)
        ones = jnp.ones((_L,), jnp.float32)

        def _zero_row(r, carry):
            @plsc.parallel_loop(0, _NL // _L, unroll=8)
            def _z(u):
                acc[pl.ds(r * _NL + u * _L, _L)] = zeros
            return carry
        lax.fori_loop(0, _CS, _zero_row, 0)

        @plsc.parallel_loop(0, _NL // _L, unroll=8)
        def _zero_cacc(u):
            cacc[pl.ds(u * _L, _L)] = zeros

        def _prep_row(r, carry):
            @plsc.parallel_loop(0, _GPR, unroll=8)
            def _grp(u):
                iv = ivf[r, pl.ds(u * _L, _L)]
                x = iv * _L + lanes
                ivf[r, pl.ds(u * _L, _L)] = x
                plsc.addupdate_scatter(cacc, [x], ones)
            return carry
        lax.fori_loop(0, _ROWS, _prep_row, 0)

        def _pair(kk, carry):
            ch0 = 2 * kk
            for half, fa, fb, sema, semb in (
                (0, fa0, fb0, sa0, sb0),
                (1, fa1, fb1, sa1, sb1),
            ):
                rbase = half * _HR
                pltpu.make_async_copy(
                    feat_hbm.at[b, ch0, pl.ds(r0 + rbase, _HR), :], fa, sema
                ).wait()
                pltpu.make_async_copy(
                    feat_hbm.at[b, ch0 + 1, pl.ds(r0 + rbase, _HR), :], fb,
                    semb,
                ).wait()
                acc0 = acc.at[pl.ds(ch0 * _NL, _NL)]
                acc1 = acc.at[pl.ds((ch0 + 1) * _NL, _NL)]

                def _row(r, inner, fa=fa, fb=fb, rbase=rbase,
                         acc0=acc0, acc1=acc1):
                    @plsc.parallel_loop(0, _GPR, unroll=8)
                    def _grp(u):
                        x = ivf[r + rbase, pl.ds(u * _L, _L)]
                        va = fa[r, pl.ds(u * _L, _L)]
                        vb = fb[r, pl.ds(u * _L, _L)]
                        plsc.addupdate_scatter(acc0, [x], va)
                        plsc.addupdate_scatter(acc1, [x], vb)
                    return inner
                lax.fori_loop(0, _HR, _row, 0)

                @pl.when(ch0 + 2 < _CS)
                def _prefetch(ch0=ch0, rbase=rbase, fa=fa, fb=fb,
                              sema=sema, semb=semb):
                    pltpu.async_copy(
                        feat_hbm.at[b, ch0 + 2, pl.ds(r0 + rbase, _HR), :],
                        fa, sema,
                    )
                    pltpu.async_copy(
                        feat_hbm.at[b, ch0 + 3, pl.ds(r0 + rbase, _HR), :],
                        fb, semb,
                    )
            return carry
        lax.fori_loop(0, _CS // 2, _pair, 0)

        pltpu.sync_copy(acc, psum_hbm.at[wid])
        pltpu.sync_copy(cacc, pcnt_hbm.at[wid])

    return seg_kernel(feat, idx)


def _tc_segment_sums(feat, idx):
    def body(f_ref, i_ref, out_ref):
        acc = jnp.zeros((_N, _CT), jnp.float32)
        seg = lax.broadcasted_iota(jnp.int32, (_N, _W), 0)
        for hr in range(_HB):
            xr = f_ref[0, :, hr, :]
            ir = i_ref[0, hr, :]
            oh = (jnp.broadcast_to(ir[None, :], (_N, _W)) == seg)
            acc = acc + lax.dot_general(
                oh.astype(jnp.float32), xr, (((1,), (1,)), ((), ())),
                precision=lax.Precision.HIGHEST)

        @pl.when(pl.program_id(1) == 0)
        def _init():
            out_ref[0] = acc

        @pl.when(pl.program_id(1) != 0)
        def _accum():
            out_ref[0] += acc

    return pl.pallas_call(
        body,
        grid=(_B, _H // _HB),
        in_specs=[
            pl.BlockSpec((1, _CT, _HB, _W), lambda b, h: (b, _CS // _CT, h, 0)),
            pl.BlockSpec((1, _HB, _W), lambda b, h: (b, h, 0)),
        ],
        out_specs=pl.BlockSpec((1, _N, _CT), lambda b, h: (b, 0, 0)),
        out_shape=jax.ShapeDtypeStruct((_B, _N, _CT), jnp.float32),
    )(feat, idx)


def _tc_finalize(psum, pcnt, tcsum):
    def body(ps_ref, pc_ref, tc_ref, sp_ref, sim_ref):
        x = jnp.sum(ps_ref[0], axis=0)
        cn = jnp.sum(pc_ref[0], axis=0)
        col = lax.broadcasted_iota(jnp.int32, (_NL, _N), 0) // _L
        seg = lax.broadcasted_iota(jnp.int32, (_NL, _N), 1)
        onehot = (col == seg).astype(jnp.float32)
        dims = (((0,), (1,)), ((), ()))
        sums_t = lax.dot_general(onehot, x, dims,
                                 precision=lax.Precision.HIGHEST)
        cnt_t = lax.dot_general(onehot, cn[None, :], dims,
                                precision=lax.Precision.HIGHEST)
        inv = 1.0 / jnp.maximum(cnt_t, 1.0)
        m_sc = sums_t * inv
        m_tc = tc_ref[0] * inv
        sp_ref[0, :, 0:_CS] = m_sc
        sp_ref[0, :, _CS:_C] = m_tc
        gdims = (((1,), (1,)), ((), ()))
        gram = (lax.dot_general(m_sc, m_sc, gdims,
                                precision=lax.Precision.HIGHEST)
                + lax.dot_general(m_tc, m_tc, gdims,
                                  precision=lax.Precision.HIGHEST))
        r = jnp.sum(m_sc * m_sc, axis=1) + jnp.sum(m_tc * m_tc, axis=1)
        sim_ref[0] = 1.0 - 0.5 * (r[:, None] + r[None, :]) + gram

    return pl.pallas_call(
        body,
        grid=(_B,),
        in_specs=[
            pl.BlockSpec((1, _SH, _CS, _NL), lambda b: (b, 0, 0, 0)),
            pl.BlockSpec((1, _SH, _NL), lambda b: (b, 0, 0)),
            pl.BlockSpec((1, _N, _CT), lambda b: (b, 0, 0)),
        ],
        out_specs=[
            pl.BlockSpec((1, _N, _C), lambda b: (b, 0, 0)),
            pl.BlockSpec((1, _N, _N), lambda b: (b, 0, 0)),
        ],
        out_shape=[
            jax.ShapeDtypeStruct((_B, _N, _C), jnp.float32),
            jax.ShapeDtypeStruct((_B, _N, _N), jnp.float32),
        ],
    )(psum, pcnt, tcsum)


def kernel(features, superpixel_indices):
    psum, pcnt = _sc_segment_sums(features, superpixel_indices)
    tcsum = _tc_segment_sums(features, superpixel_indices)
    ps = psum.reshape(_B, _SH, _CS, _NL)
    pc = pcnt.reshape(_B, _SH, _NL)
    sp, sim = _tc_finalize(ps, pc, tcsum)
    return (sp, sim)


# speedup vs baseline: 15.4698x; 1.2145x over previous
import functools

import jax
import jax.numpy as jnp
from jax import lax
from jax.experimental import pallas as pl
from jax.experimental.pallas import tpu as pltpu
from jax.experimental.pallas import tpu_sc as plsc

_B, _C, _H, _W = 4, 96, 384, 384
_N = 32
_L = 16
_NL = _N * _L
_NC, _NS = 2, 16
_NW = _NC * _NS
_SH = _NW // _B
_ROWS = _H // _SH
_HR = _ROWS // 2
_GPR = _W // _L
_CS = 48
_CT = _C - _CS
_HB = 8


def _sc_segment_sums(feat, idx):
    mesh = plsc.VectorSubcoreMesh(core_axis_name="c", subcore_axis_name="s")

    @functools.partial(
        pl.kernel,
        out_type=(
            jax.ShapeDtypeStruct((_NW, _CS * _NL), jnp.float32),
            jax.ShapeDtypeStruct((_NW, _NL), jnp.float32),
        ),
        mesh=mesh,
        scratch_types=[
            pltpu.VMEM((_ROWS, _W), jnp.int32),
            pltpu.VMEM((_HR, _W), jnp.float32),
            pltpu.VMEM((_HR, _W), jnp.float32),
            pltpu.VMEM((_HR, _W), jnp.float32),
            pltpu.VMEM((_HR, _W), jnp.float32),
            pltpu.VMEM((_CS * _NL,), jnp.float32),
            pltpu.VMEM((_NL,), jnp.float32),
            pltpu.SemaphoreType.DMA,
            pltpu.SemaphoreType.DMA,
            pltpu.SemaphoreType.DMA,
            pltpu.SemaphoreType.DMA,
            pltpu.SemaphoreType.DMA,
        ],
        compiler_params=pltpu.CompilerParams(needs_layout_passes=False),
    )
    def seg_kernel(feat_hbm, idx_hbm, psum_hbm, pcnt_hbm,
                   ivf, fa0, fb0, fa1, fb1, acc, cacc,
                   sa0, sb0, sa1, sb1, isem):
        wid = lax.axis_index("s") * _NC + lax.axis_index("c")
        b = wid // _SH
        sh = wid % _SH
        r0 = sh * _ROWS

        pltpu.async_copy(idx_hbm.at[b, pl.ds(r0, _ROWS), :], ivf, isem).wait()
        pltpu.async_copy(feat_hbm.at[b, 0, pl.ds(r0, _HR), :], fa0, sa0)
        pltpu.async_copy(feat_hbm.at[b, 1, pl.ds(r0, _HR), :], fb0, sb0)
        pltpu.async_copy(feat_hbm.at[b, 0, pl.ds(r0 + _HR, _HR), :], fa1, sa1)
        pltpu.async_copy(feat_hbm.at[b, 1, pl.ds(r0 + _HR, _HR), :], fb1, sb1)

        lanes = lax.iota(jnp.int32, _L)
        zeros = jnp.zeros((_L,), jnp.float32)
        ones = jnp.ones((_L,), jnp.float32)

        def _zero_row(r, carry):
            @plsc.parallel_loop(0, _NL // _L, unroll=8)
            def _z(u):
                acc[pl.ds(r * _NL + u * _L, _L)] = zeros
            return carry
        lax.fori_loop(0, _CS, _zero_row, 0)

        @plsc.parallel_loop(0, _NL // _L, unroll=8)
        def _zero_cacc(u):
            cacc[pl.ds(u * _L, _L)] = zeros

        def _prep_row(r, carry):
            @plsc.parallel_loop(0, _GPR, unroll=8)
            def _grp(u):
                iv = ivf[r, pl.ds(u * _L, _L)]
                x = iv * _L + lanes
                ivf[r, pl.ds(u * _L, _L)] = x
                plsc.addupdate_scatter(cacc, [x], ones)
            return carry
        lax.fori_loop(0, _ROWS, _prep_row, 0)

        def _pair(kk, carry):
            ch0 = 2 * kk
            for half, fa, fb, sema, semb in (
                (0, fa0, fb0, sa0, sb0),
                (1, fa1, fb1, sa1, sb1),
            ):
                rbase = half * _HR
                pltpu.make_async_copy(
                    feat_hbm.at[b, ch0, pl.ds(r0 + rbase, _HR), :], fa, sema
                ).wait()
                pltpu.make_async_copy(
                    feat_hbm.at[b, ch0 + 1, pl.ds(r0 + rbase, _HR), :], fb,
                    semb,
                ).wait()
                acc0 = acc.at[pl.ds(ch0 * _NL, _NL)]
                acc1 = acc.at[pl.ds((ch0 + 1) * _NL, _NL)]

                def _row(r, inner, fa=fa, fb=fb, rbase=rbase,
                         acc0=acc0, acc1=acc1):
                    @plsc.parallel_loop(0, _GPR, unroll=8)
                    def _grp(u):
                        x = ivf[r + rbase, pl.ds(u * _L, _L)]
                        va = fa[r, pl.ds(u * _L, _L)]
                        vb = fb[r, pl.ds(u * _L, _L)]
                        plsc.addupdate_scatter(acc0, [x], va)
                        plsc.addupdate_scatter(acc1, [x], vb)
                    return inner
                lax.fori_loop(0, _HR, _row, 0)

                @pl.when(ch0 + 2 < _CS)
                def _prefetch(ch0=ch0, rbase=rbase, fa=fa, fb=fb,
                              sema=sema, semb=semb):
                    pltpu.async_copy(
                        feat_hbm.at[b, ch0 + 2, pl.ds(r0 + rbase, _HR), :],
                        fa, sema,
                    )
                    pltpu.async_copy(
                        feat_hbm.at[b, ch0 + 3, pl.ds(r0 + rbase, _HR), :],
                        fb, semb,
                    )
            return carry
        lax.fori_loop(0, _CS // 2, _pair, 0)

        pltpu.sync_copy(acc, psum_hbm.at[wid])
        pltpu.sync_copy(cacc, pcnt_hbm.at[wid])

    return seg_kernel(feat, idx)


def _tc_segment_sums(feat, idx):
    def body(f_ref, i_ref, out_ref):
        acc = jnp.zeros((_N, _CT), jnp.float32)
        seg = lax.broadcasted_iota(jnp.int32, (_N, _HB * _W), 0)
        ir = i_ref[0, 0]
        oh = (jnp.broadcast_to(ir[None, :], (_N, _HB * _W)) == seg)
        ohf = oh.astype(jnp.float32)
        for hr in range(_HB):
            xr = f_ref[0, :, hr, :]
            ohr = lax.slice(ohf, (0, hr * _W), (_N, (hr + 1) * _W))
            acc = acc + lax.dot_general(
                ohr, xr, (((1,), (1,)), ((), ())))

        @pl.when(pl.program_id(1) == 0)
        def _init():
            out_ref[0] = acc

        @pl.when(pl.program_id(1) != 0)
        def _accum():
            out_ref[0] += acc

    return pl.pallas_call(
        body,
        grid=(_B, _H // _HB),
        in_specs=[
            pl.BlockSpec((1, _CT, _HB, _W), lambda b, h: (b, _CS // _CT, h, 0)),
            pl.BlockSpec((1, 1, _HB * _W),
                         lambda b, h: (b * (_H // _HB) + h, 0, 0)),
        ],
        out_specs=pl.BlockSpec((1, _N, _CT), lambda b, h: (b, 0, 0)),
        out_shape=jax.ShapeDtypeStruct((_B, _N, _CT), jnp.float32),
    )(feat, idx.reshape(_B * (_H // _HB), 1, _HB * _W))


def _tc_finalize(psum, pcnt, tcsum):
    def body(ps_ref, pc_ref, tc_ref, sp_ref, sim_ref):
        x = jnp.sum(ps_ref[0], axis=0)
        cn = jnp.sum(pc_ref[0], axis=0)
        col = lax.broadcasted_iota(jnp.int32, (_NL, _N), 0) // _L
        seg = lax.broadcasted_iota(jnp.int32, (_NL, _N), 1)
        onehot = (col == seg).astype(jnp.float32)
        dims = (((0,), (1,)), ((), ()))
        sums_t = lax.dot_general(onehot, x, dims,
                                 precision=lax.Precision.HIGHEST)
        cnt_t = lax.dot_general(onehot, cn[None, :], dims,
                                precision=lax.Precision.HIGHEST)
        inv = 1.0 / jnp.maximum(cnt_t, 1.0)
        m_sc = sums_t * inv
        m_tc = tc_ref[0] * inv
        sp_ref[0, :, 0:_CS] = m_sc
        sp_ref[0, :, _CS:_C] = m_tc
        gdims = (((1,), (1,)), ((), ()))
        gram = (lax.dot_general(m_sc, m_sc, gdims,
                                precision=lax.Precision.HIGHEST)
                + lax.dot_general(m_tc, m_tc, gdims,
                                  precision=lax.Precision.HIGHEST))
        r = jnp.sum(m_sc * m_sc, axis=1) + jnp.sum(m_tc * m_tc, axis=1)
        sim_ref[0] = 1.0 - 0.5 * (r[:, None] + r[None, :]) + gram

    return pl.pallas_call(
        body,
        grid=(_B,),
        in_specs=[
            pl.BlockSpec((1, _SH, _CS, _NL), lambda b: (b, 0, 0, 0)),
            pl.BlockSpec((1, _SH, _NL), lambda b: (b, 0, 0)),
            pl.BlockSpec((1, _N, _CT), lambda b: (b, 0, 0)),
        ],
        out_specs=[
            pl.BlockSpec((1, _N, _C), lambda b: (b, 0, 0)),
            pl.BlockSpec((1, _N, _N), lambda b: (b, 0, 0)),
        ],
        out_shape=[
            jax.ShapeDtypeStruct((_B, _N, _C), jnp.float32),
            jax.ShapeDtypeStruct((_B, _N, _N), jnp.float32),
        ],
    )(psum, pcnt, tcsum)


def kernel(features, superpixel_indices):
    psum, pcnt = _sc_segment_sums(features, superpixel_indices)
    tcsum = _tc_segment_sums(features, superpixel_indices)
    ps = psum.reshape(_B, _SH, _CS, _NL)
    pc = pcnt.reshape(_B, _SH, _NL)
    sp, sim = _tc_finalize(ps, pc, tcsum)
    return (sp, sim)
